# Initial kernel scaffold; baseline (speedup 1.0000x reference)
import jax
import jax.numpy as jnp
from jax.experimental import pallas as pl


def kernel(x, edge_index, coordinates, W0, b0, W1, b1):
    raise NotImplementedError("write your pallas kernel here")


# speedup vs baseline: 9.2848x; 9.2848x over previous
import functools

import jax
import jax.numpy as jnp
from jax import lax
from jax.experimental import pallas as pl
from jax.experimental.pallas import tpu as pltpu
from jax.experimental.pallas import tpu_sc as plsc

N = 2048
D = 128
E = 65536
NW = 32
ROWS = 32
NBLK = N // ROWS
BPW = NBLK // NW
CHUNK = 8192
NCH = E // CHUNK


@functools.cache
def _sc_build_fn():
    mesh = plsc.VectorSubcoreMesh(core_axis_name="c", subcore_axis_name="s")

    @functools.partial(
        pl.kernel,
        out_type=(
            jax.ShapeDtypeStruct((N * N,), jnp.float32),
            jax.ShapeDtypeStruct((N,), jnp.float32),
        ),
        mesh=mesh,
        compiler_params=pltpu.CompilerParams(needs_layout_passes=False),
        scratch_types=(
            pltpu.VMEM((ROWS * N,), jnp.float32),
            pltpu.VMEM((ROWS,), jnp.float32),
            pltpu.VMEM((CHUNK,), jnp.int32),
            pltpu.VMEM((CHUNK,), jnp.int32),
        ),
    )
    def _sc_build(dst_hbm, src_hbm, c_hbm, deg_hbm, acc, dega, dbuf, sbuf):
        wid = lax.axis_index("s") * 2 + lax.axis_index("c")
        ones = jnp.ones((16,), jnp.float32)
        zeros = jnp.zeros((16,), jnp.float32)
        for p in range(BPW):
            blk = wid + NW * p
            base = blk * ROWS

            def zbody(j, _):
                acc[pl.ds(j * 16, 16)] = zeros
                return 0

            lax.fori_loop(0, ROWS * N // 16, zbody, 0)
            dega[pl.ds(0, 16)] = zeros
            dega[pl.ds(16, 16)] = zeros

            for ch in range(NCH):
                pltpu.sync_copy(dst_hbm.at[pl.ds(ch * CHUNK, CHUNK)], dbuf)
                pltpu.sync_copy(src_hbm.at[pl.ds(ch * CHUNK, CHUNK)], sbuf)

                def ebody(i, _):
                    dv = dbuf[pl.ds(i * 16, 16)]
                    sv = sbuf[pl.ds(i * 16, 16)]
                    loc = dv - base
                    m = (loc >= 0) & (loc < ROWS)
                    locc = lax.bitwise_and(loc, ROWS - 1)
                    plsc.addupdate_scatter(acc, [locc * N + sv], ones, mask=m)
                    plsc.addupdate_scatter(dega, [locc], ones, mask=m)
                    return 0

                lax.fori_loop(0, CHUNK // 16, ebody, 0)

            for h in range(ROWS // 16):
                r = lax.iota(jnp.int32, 16) + h * 16
                plsc.addupdate_scatter(acc, [r * N + base + r], ones)

            pltpu.sync_copy(acc, c_hbm.at[pl.ds(base * N, ROWS * N)])
            pltpu.sync_copy(dega, deg_hbm.at[pl.ds(base, ROWS)])

    return _sc_build


BR = 256
_PREC = lax.Precision.HIGHEST


def _tc1_body(x_ref, w0_ref, degf_ref, degb_ref, c_ref, w1_ref, b0_ref,
              t1_ref, t0_scr):
    g = pl.program_id(0)

    @pl.when(g == 0)
    def _():
        dis = lax.rsqrt(degf_ref[...] + 1.0)
        t0_scr[...] = dis * jnp.dot(
            x_ref[...], w0_ref[...],
            preferred_element_type=jnp.float32, precision=_PREC)

    disb = lax.rsqrt(degb_ref[...] + 1.0)
    m = jnp.dot(c_ref[...], t0_scr[...],
                preferred_element_type=jnp.float32, precision=_PREC)
    h1 = jnp.maximum(disb * m + b0_ref[...], 0.0)
    t1_ref[...] = jnp.dot(h1, w1_ref[...],
                          preferred_element_type=jnp.float32,
                          precision=_PREC) * disb


def _tc2_body(c_ref, t1_ref, degb_ref, b1_ref, out_ref):
    disb = lax.rsqrt(degb_ref[...] + 1.0)
    out_ref[...] = disb * jnp.dot(
        c_ref[...], t1_ref[...],
        preferred_element_type=jnp.float32, precision=_PREC) + b1_ref[...]


def _gcn_stack(x, C, deg, W0, b0, W1, b1):
    deg2 = deg.reshape(N, 1)
    t1 = pl.pallas_call(
        _tc1_body,
        grid=(N // BR,),
        in_specs=[
            pl.BlockSpec((N, D), lambda g: (0, 0)),
            pl.BlockSpec((D, D), lambda g: (0, 0)),
            pl.BlockSpec((N, 1), lambda g: (0, 0)),
            pl.BlockSpec((BR, 1), lambda g: (g, 0)),
            pl.BlockSpec((BR, N), lambda g: (g, 0)),
            pl.BlockSpec((D, D), lambda g: (0, 0)),
            pl.BlockSpec((1, D), lambda g: (0, 0)),
        ],
        out_specs=pl.BlockSpec((BR, D), lambda g: (g, 0)),
        out_shape=jax.ShapeDtypeStruct((N, D), jnp.float32),
        scratch_shapes=[pltpu.VMEM((N, D), jnp.float32)],
    )(x, W0, deg2, deg2, C, W1, b0.reshape(1, D))
    out = pl.pallas_call(
        _tc2_body,
        grid=(N // BR,),
        in_specs=[
            pl.BlockSpec((BR, N), lambda g: (g, 0)),
            pl.BlockSpec((N, D), lambda g: (0, 0)),
            pl.BlockSpec((BR, 1), lambda g: (g, 0)),
            pl.BlockSpec((1, D), lambda g: (0, 0)),
        ],
        out_specs=pl.BlockSpec((BR, D), lambda g: (g, 0)),
        out_shape=jax.ShapeDtypeStruct((N, D), jnp.float32),
    )(C, t1, deg2, b1.reshape(1, D))
    return out


def kernel(x, edge_index, coordinates, W0, b0, W1, b1):
    ei = edge_index.astype(jnp.int32)
    src = ei[0]
    dst = ei[1]
    c_flat, deg = _sc_build_fn()(dst, src)
    C = c_flat.reshape(N, N)
    out = _gcn_stack(x, C, deg, W0, b0, W1, b1)
    labels = jax.random.randint(jax.random.key(42), (x.shape[0],), 0, 3)
    return (out, labels)


# speedup vs baseline: 13.1960x; 1.4212x over previous
import functools

import jax
import jax.numpy as jnp
from jax import lax
from jax.experimental import pallas as pl
from jax.experimental.pallas import tpu as pltpu
from jax.experimental.pallas import tpu_sc as plsc

N = 2048
D = 128
E = 65536
NW = 32
ROWS = 32
NBLK = N // ROWS
BPW = NBLK // NW
CHUNK = 8192
NCH = E // CHUNK


@functools.cache
def _sc_build_fn():
    mesh = plsc.VectorSubcoreMesh(core_axis_name="c", subcore_axis_name="s")

    @functools.partial(
        pl.kernel,
        out_type=(
            jax.ShapeDtypeStruct((N * N,), jnp.float32),
            jax.ShapeDtypeStruct((N,), jnp.float32),
        ),
        mesh=mesh,
        compiler_params=pltpu.CompilerParams(needs_layout_passes=False),
        scratch_types=(
            pltpu.VMEM((ROWS * N,), jnp.float32),
            pltpu.VMEM((ROWS,), jnp.float32),
            pltpu.VMEM((CHUNK,), jnp.int32),
            pltpu.VMEM((CHUNK,), jnp.int32),
            pltpu.VMEM((CHUNK,), jnp.int32),
            pltpu.VMEM((CHUNK,), jnp.int32),
            pltpu.SemaphoreType.DMA,
            pltpu.SemaphoreType.DMA,
        ),
    )
    def _sc_build(dst_hbm, src_hbm, c_hbm, deg_hbm, acc, dega,
                  dbuf0, sbuf0, dbuf1, sbuf1, sem0, sem1):
        wid = lax.axis_index("s") * 2 + lax.axis_index("c")
        slots = ((dbuf0, sbuf0, sem0), (dbuf1, sbuf1, sem1))
        ones = jnp.ones((16,), jnp.float32)
        zeros = jnp.zeros((16,), jnp.float32)

        def start(gch, slot):
            ch = gch % NCH
            db, sb, sem = slots[slot]
            return (
                pltpu.async_copy(dst_hbm.at[pl.ds(ch * CHUNK, CHUNK)],
                                 db, sem),
                pltpu.async_copy(src_hbm.at[pl.ds(ch * CHUNK, CHUNK)],
                                 sb, sem),
            )

        pending = {0: start(0, 0)}
        for p in range(BPW):
            blk = wid + NW * p
            base = blk * ROWS

            def zbody(j, _):
                acc[pl.ds(j * 16, 16)] = zeros
                return 0

            lax.fori_loop(0, ROWS * N // 16, zbody, 0)
            dega[pl.ds(0, 16)] = zeros
            dega[pl.ds(16, 16)] = zeros

            for ch in range(NCH):
                gch = p * NCH + ch
                slot = gch % 2
                for h in pending.pop(gch):
                    h.wait()
                if gch + 1 < BPW * NCH:
                    pending[gch + 1] = start(gch + 1, 1 - slot)
                db, sb, _ = slots[slot]

                @plsc.parallel_loop(0, CHUNK // 16, unroll=4)
                def ebody(i):
                    dv = db[pl.ds(i * 16, 16)]
                    sv = sb[pl.ds(i * 16, 16)]
                    loc = dv - base
                    m = (loc >= 0) & (loc < ROWS)
                    locc = lax.bitwise_and(loc, ROWS - 1)
                    plsc.addupdate_scatter(acc, [locc * N + sv], ones, mask=m)
                    plsc.addupdate_scatter(dega, [locc], ones, mask=m)

            for h in range(ROWS // 16):
                r = lax.iota(jnp.int32, 16) + h * 16
                plsc.addupdate_scatter(acc, [r * N + base + r], ones)

            pltpu.sync_copy(acc, c_hbm.at[pl.ds(base * N, ROWS * N)])
            pltpu.sync_copy(dega, deg_hbm.at[pl.ds(base, ROWS)])

    return _sc_build


BR = 256
_PREC = lax.Precision.HIGHEST


def _tc1_body(x_ref, w0_ref, degf_ref, degb_ref, c_ref, w1_ref, b0_ref,
              t1_ref, t0_scr):
    g = pl.program_id(0)

    @pl.when(g == 0)
    def _():
        dis = lax.rsqrt(degf_ref[...] + 1.0)
        t0_scr[...] = dis * jnp.dot(
            x_ref[...], w0_ref[...],
            preferred_element_type=jnp.float32, precision=_PREC)

    disb = lax.rsqrt(degb_ref[...] + 1.0)
    m = jnp.dot(c_ref[...], t0_scr[...],
                preferred_element_type=jnp.float32, precision=_PREC)
    h1 = jnp.maximum(disb * m + b0_ref[...], 0.0)
    t1_ref[...] = jnp.dot(h1, w1_ref[...],
                          preferred_element_type=jnp.float32,
                          precision=_PREC) * disb


def _tc2_body(c_ref, t1_ref, degb_ref, b1_ref, out_ref):
    disb = lax.rsqrt(degb_ref[...] + 1.0)
    out_ref[...] = disb * jnp.dot(
        c_ref[...], t1_ref[...],
        preferred_element_type=jnp.float32, precision=_PREC) + b1_ref[...]


def _gcn_stack(x, C, deg, W0, b0, W1, b1):
    deg2 = deg.reshape(N, 1)
    t1 = pl.pallas_call(
        _tc1_body,
        grid=(N // BR,),
        in_specs=[
            pl.BlockSpec((N, D), lambda g: (0, 0)),
            pl.BlockSpec((D, D), lambda g: (0, 0)),
            pl.BlockSpec((N, 1), lambda g: (0, 0)),
            pl.BlockSpec((BR, 1), lambda g: (g, 0)),
            pl.BlockSpec((BR, N), lambda g: (g, 0)),
            pl.BlockSpec((D, D), lambda g: (0, 0)),
            pl.BlockSpec((1, D), lambda g: (0, 0)),
        ],
        out_specs=pl.BlockSpec((BR, D), lambda g: (g, 0)),
        out_shape=jax.ShapeDtypeStruct((N, D), jnp.float32),
        scratch_shapes=[pltpu.VMEM((N, D), jnp.float32)],
    )(x, W0, deg2, deg2, C, W1, b0.reshape(1, D))
    out = pl.pallas_call(
        _tc2_body,
        grid=(N // BR,),
        in_specs=[
            pl.BlockSpec((BR, N), lambda g: (g, 0)),
            pl.BlockSpec((N, D), lambda g: (0, 0)),
            pl.BlockSpec((BR, 1), lambda g: (g, 0)),
            pl.BlockSpec((1, D), lambda g: (0, 0)),
        ],
        out_specs=pl.BlockSpec((BR, D), lambda g: (g, 0)),
        out_shape=jax.ShapeDtypeStruct((N, D), jnp.float32),
    )(C, t1, deg2, b1.reshape(1, D))
    return out


def kernel(x, edge_index, coordinates, W0, b0, W1, b1):
    ei = edge_index.astype(jnp.int32)
    src = ei[0]
    dst = ei[1]
    c_flat, deg = _sc_build_fn()(dst, src)
    C = c_flat.reshape(N, N)
    out = _gcn_stack(x, C, deg, W0, b0, W1, b1)
    labels = jax.random.randint(jax.random.key(42), (x.shape[0],), 0, 3)
    return (out, labels)


# speedup vs baseline: 14.3628x; 1.0884x over previous
import functools

import jax
import jax.numpy as jnp
from jax import lax
from jax.experimental import pallas as pl
from jax.experimental.pallas import tpu as pltpu
from jax.experimental.pallas import tpu_sc as plsc

N = 2048
D = 128
E = 65536
NW = 32
ROWS = 32
NBLK = N // ROWS
BPW = NBLK // NW
CHUNK = 8192
NCH = E // CHUNK


@functools.cache
def _sc_build_fn():
    mesh = plsc.VectorSubcoreMesh(core_axis_name="c", subcore_axis_name="s")

    @functools.partial(
        pl.kernel,
        out_type=(
            jax.ShapeDtypeStruct((N * N,), jnp.float32),
            jax.ShapeDtypeStruct((N,), jnp.float32),
        ),
        mesh=mesh,
        compiler_params=pltpu.CompilerParams(needs_layout_passes=False),
        scratch_types=(
            pltpu.VMEM((ROWS * N,), jnp.float32),
            pltpu.VMEM((ROWS,), jnp.float32),
            pltpu.VMEM((CHUNK,), jnp.int32),
            pltpu.VMEM((CHUNK,), jnp.int32),
            pltpu.VMEM((CHUNK,), jnp.int32),
            pltpu.VMEM((CHUNK,), jnp.int32),
            pltpu.SemaphoreType.DMA,
            pltpu.SemaphoreType.DMA,
        ),
    )
    def _sc_build(dst_hbm, src_hbm, c_hbm, deg_hbm, acc, dega,
                  dbuf0, sbuf0, dbuf1, sbuf1, sem0, sem1):
        wid = lax.axis_index("s") * 2 + lax.axis_index("c")
        slots = ((dbuf0, sbuf0, sem0), (dbuf1, sbuf1, sem1))
        ones = jnp.ones((16,), jnp.float32)
        zeros = jnp.zeros((16,), jnp.float32)

        def start(gch, slot):
            ch = gch % NCH
            db, sb, sem = slots[slot]
            return (
                pltpu.async_copy(dst_hbm.at[pl.ds(ch * CHUNK, CHUNK)],
                                 db, sem),
                pltpu.async_copy(src_hbm.at[pl.ds(ch * CHUNK, CHUNK)],
                                 sb, sem),
            )

        pending = {0: start(0, 0)}
        for p in range(BPW):
            blk = wid + NW * p
            base = blk * ROWS

            def zbody(j, _):
                acc[pl.ds(j * 16, 16)] = zeros
                return 0

            lax.fori_loop(0, ROWS * N // 16, zbody, 0)
            dega[pl.ds(0, 16)] = zeros
            dega[pl.ds(16, 16)] = zeros

            for ch in range(NCH):
                gch = p * NCH + ch
                slot = gch % 2
                for h in pending.pop(gch):
                    h.wait()
                if gch + 1 < BPW * NCH:
                    pending[gch + 1] = start(gch + 1, 1 - slot)
                db, sb, _ = slots[slot]

                @plsc.parallel_loop(0, CHUNK // 16, unroll=4)
                def ebody(i):
                    dv = db[pl.ds(i * 16, 16)]
                    sv = sb[pl.ds(i * 16, 16)]
                    loc = dv - base
                    m = (loc >= 0) & (loc < ROWS)
                    locc = lax.bitwise_and(loc, ROWS - 1)
                    plsc.addupdate_scatter(acc, [locc * N + sv], ones, mask=m)
                    plsc.addupdate_scatter(dega, [locc], ones, mask=m)

            for h in range(ROWS // 16):
                r = lax.iota(jnp.int32, 16) + h * 16
                plsc.addupdate_scatter(acc, [r * N + base + r], ones)

            pltpu.sync_copy(acc, c_hbm.at[pl.ds(base * N, ROWS * N)])
            pltpu.sync_copy(dega, deg_hbm.at[pl.ds(base, ROWS)])

    return _sc_build


BR = 256
_PREC = lax.Precision.HIGHEST


def _cdot(c_ref, t):
    cb = c_ref[...].astype(jnp.bfloat16)
    th = t.astype(jnp.bfloat16)
    tl = (t - th.astype(jnp.float32)).astype(jnp.bfloat16)
    return (jnp.dot(cb, th, preferred_element_type=jnp.float32)
            + jnp.dot(cb, tl, preferred_element_type=jnp.float32))


def _tc1_body(x_ref, w0_ref, degf_ref, degb_ref, c_ref, w1_ref, b0_ref,
              t1_ref, t0_scr):
    g = pl.program_id(0)

    @pl.when(g == 0)
    def _():
        dis = lax.rsqrt(degf_ref[...] + 1.0)
        t0_scr[...] = dis * jnp.dot(
            x_ref[...], w0_ref[...],
            preferred_element_type=jnp.float32, precision=_PREC)

    disb = lax.rsqrt(degb_ref[...] + 1.0)
    m = _cdot(c_ref, t0_scr[...])
    h1 = jnp.maximum(disb * m + b0_ref[...], 0.0)
    t1_ref[...] = jnp.dot(h1, w1_ref[...],
                          preferred_element_type=jnp.float32,
                          precision=_PREC) * disb


def _tc2_body(c_ref, t1_ref, degb_ref, b1_ref, out_ref):
    disb = lax.rsqrt(degb_ref[...] + 1.0)
    out_ref[...] = disb * _cdot(c_ref, t1_ref[...]) + b1_ref[...]


def _gcn_stack(x, C, deg, W0, b0, W1, b1):
    deg2 = deg.reshape(N, 1)
    t1 = pl.pallas_call(
        _tc1_body,
        grid=(N // BR,),
        in_specs=[
            pl.BlockSpec((N, D), lambda g: (0, 0)),
            pl.BlockSpec((D, D), lambda g: (0, 0)),
            pl.BlockSpec((N, 1), lambda g: (0, 0)),
            pl.BlockSpec((BR, 1), lambda g: (g, 0)),
            pl.BlockSpec((BR, N), lambda g: (g, 0)),
            pl.BlockSpec((D, D), lambda g: (0, 0)),
            pl.BlockSpec((1, D), lambda g: (0, 0)),
        ],
        out_specs=pl.BlockSpec((BR, D), lambda g: (g, 0)),
        out_shape=jax.ShapeDtypeStruct((N, D), jnp.float32),
        scratch_shapes=[pltpu.VMEM((N, D), jnp.float32)],
    )(x, W0, deg2, deg2, C, W1, b0.reshape(1, D))
    out = pl.pallas_call(
        _tc2_body,
        grid=(N // BR,),
        in_specs=[
            pl.BlockSpec((BR, N), lambda g: (g, 0)),
            pl.BlockSpec((N, D), lambda g: (0, 0)),
            pl.BlockSpec((BR, 1), lambda g: (g, 0)),
            pl.BlockSpec((1, D), lambda g: (0, 0)),
        ],
        out_specs=pl.BlockSpec((BR, D), lambda g: (g, 0)),
        out_shape=jax.ShapeDtypeStruct((N, D), jnp.float32),
    )(C, t1, deg2, b1.reshape(1, D))
    return out


def kernel(x, edge_index, coordinates, W0, b0, W1, b1):
    ei = edge_index.astype(jnp.int32)
    src = ei[0]
    dst = ei[1]
    c_flat, deg = _sc_build_fn()(dst, src)
    C = c_flat.reshape(N, N)
    out = _gcn_stack(x, C, deg, W0, b0, W1, b1)
    labels = jax.random.randint(jax.random.key(42), (x.shape[0],), 0, 3)
    return (out, labels)


# speedup vs baseline: 14.4367x; 1.0051x over previous
import functools

import jax
import jax.numpy as jnp
from jax import lax
from jax.experimental import pallas as pl
from jax.experimental.pallas import tpu as pltpu
from jax.experimental.pallas import tpu_sc as plsc

N = 2048
D = 128
E = 65536
NW = 32
ROWS = 32
NBLK = N // ROWS
BPW = NBLK // NW
CHUNK = 8192
NCH = E // CHUNK


@functools.cache
def _sc_build_fn():
    mesh = plsc.VectorSubcoreMesh(core_axis_name="c", subcore_axis_name="s")

    @functools.partial(
        pl.kernel,
        out_type=(
            jax.ShapeDtypeStruct((N * N,), jnp.float32),
            jax.ShapeDtypeStruct((N,), jnp.float32),
        ),
        mesh=mesh,
        compiler_params=pltpu.CompilerParams(needs_layout_passes=False),
        scratch_types=(
            pltpu.VMEM((ROWS * N,), jnp.float32),
            pltpu.VMEM((ROWS,), jnp.float32),
            pltpu.VMEM((CHUNK,), jnp.int32),
            pltpu.VMEM((CHUNK,), jnp.int32),
            pltpu.VMEM((CHUNK,), jnp.int32),
            pltpu.VMEM((CHUNK,), jnp.int32),
            pltpu.SemaphoreType.DMA,
            pltpu.SemaphoreType.DMA,
        ),
    )
    def _sc_build(dst_hbm, src_hbm, c_hbm, deg_hbm, acc, dega,
                  dbuf0, sbuf0, dbuf1, sbuf1, sem0, sem1):
        wid = lax.axis_index("s") * 2 + lax.axis_index("c")
        slots = ((dbuf0, sbuf0, sem0), (dbuf1, sbuf1, sem1))
        ones = jnp.ones((16,), jnp.float32)
        zeros = jnp.zeros((16,), jnp.float32)

        def start(gch, slot):
            ch = gch % NCH
            db, sb, sem = slots[slot]
            return (
                pltpu.async_copy(dst_hbm.at[pl.ds(ch * CHUNK, CHUNK)],
                                 db, sem),
                pltpu.async_copy(src_hbm.at[pl.ds(ch * CHUNK, CHUNK)],
                                 sb, sem),
            )

        pending = {0: start(0, 0)}
        for p in range(BPW):
            blk = wid + NW * p
            base = blk * ROWS

            def zbody(j, _):
                acc[pl.ds(j * 16, 16)] = zeros
                return 0

            lax.fori_loop(0, ROWS * N // 16, zbody, 0)
            dega[pl.ds(0, 16)] = zeros
            dega[pl.ds(16, 16)] = zeros

            for ch in range(NCH):
                gch = p * NCH + ch
                slot = gch % 2
                for h in pending.pop(gch):
                    h.wait()
                if gch + 1 < BPW * NCH:
                    pending[gch + 1] = start(gch + 1, 1 - slot)
                db, sb, _ = slots[slot]

                @plsc.parallel_loop(0, CHUNK // 16, unroll=4)
                def ebody(i):
                    dv = db[pl.ds(i * 16, 16)]
                    sv = sb[pl.ds(i * 16, 16)]
                    loc = dv - base
                    m = (loc >= 0) & (loc < ROWS)
                    locc = lax.bitwise_and(loc, ROWS - 1)
                    plsc.addupdate_scatter(acc, [locc * N + sv], ones, mask=m)
                    plsc.addupdate_scatter(dega, [locc], ones, mask=m)

            for h in range(ROWS // 16):
                r = lax.iota(jnp.int32, 16) + h * 16
                plsc.addupdate_scatter(acc, [r * N + base + r], ones)

            pltpu.sync_copy(acc, c_hbm.at[pl.ds(base * N, ROWS * N)])
            pltpu.sync_copy(dega, deg_hbm.at[pl.ds(base, ROWS)])

    return _sc_build


BR = 256
_PREC = lax.Precision.HIGHEST


def _cdot(c_ref, t):
    th = t.astype(jnp.bfloat16).astype(jnp.float32)
    tl = t - th
    return (jnp.dot(c_ref[...], th, preferred_element_type=jnp.float32)
            + jnp.dot(c_ref[...], tl, preferred_element_type=jnp.float32))


def _tc_body(x_ref, w0_ref, deg_ref, c_ref, w1_ref, b0_ref, b1_ref, out_ref):
    dis = lax.rsqrt(deg_ref[...] + 1.0)
    t0 = dis * jnp.dot(x_ref[...], w0_ref[...],
                       preferred_element_type=jnp.float32, precision=_PREC)
    h1 = jnp.maximum(dis * _cdot(c_ref, t0) + b0_ref[...], 0.0)
    t1 = jnp.dot(h1, w1_ref[...],
                 preferred_element_type=jnp.float32, precision=_PREC) * dis
    out_ref[...] = dis * _cdot(c_ref, t1) + b1_ref[...]


def _gcn_stack(x, C, deg, W0, b0, W1, b1):
    deg2 = deg.reshape(N, 1)
    return pl.pallas_call(
        _tc_body,
        out_shape=jax.ShapeDtypeStruct((N, D), jnp.float32),
        compiler_params=pltpu.CompilerParams(
            vmem_limit_bytes=100 * 1024 * 1024),
    )(x, W0, deg2, C, W1, b0.reshape(1, D), b1.reshape(1, D))


def kernel(x, edge_index, coordinates, W0, b0, W1, b1):
    ei = edge_index.astype(jnp.int32)
    src = ei[0]
    dst = ei[1]
    c_flat, deg = _sc_build_fn()(dst, src)
    C = c_flat.reshape(N, N)
    out = _gcn_stack(x, C, deg, W0, b0, W1, b1)
    labels = jax.random.randint(jax.random.key(42), (x.shape[0],), 0, 3)
    return (out, labels)


# speedup vs baseline: 18.2296x; 1.2627x over previous
import functools

import jax
import jax.numpy as jnp
from jax import lax
from jax.experimental import pallas as pl
from jax.experimental.pallas import tpu as pltpu
from jax.experimental.pallas import tpu_sc as plsc

N = 2048
D = 128
E = 65536
NW = 32
ROWS = 32
NBLK = N // ROWS
BPW = NBLK // NW
CHUNK = 8192
NCH = E // CHUNK


@functools.cache
def _sc_build_fn():
    mesh = plsc.VectorSubcoreMesh(core_axis_name="c", subcore_axis_name="s")

    @functools.partial(
        pl.kernel,
        out_type=(
            jax.ShapeDtypeStruct((N, N), jnp.float32),
            jax.ShapeDtypeStruct((N,), jnp.float32),
        ),
        mesh=mesh,
        compiler_params=pltpu.CompilerParams(needs_layout_passes=False),
        scratch_types=(
            pltpu.VMEM((ROWS, N), jnp.float32),
            pltpu.VMEM((ROWS,), jnp.float32),
            pltpu.VMEM((CHUNK,), jnp.int32),
            pltpu.VMEM((CHUNK,), jnp.int32),
            pltpu.VMEM((CHUNK,), jnp.int32),
            pltpu.VMEM((CHUNK,), jnp.int32),
            pltpu.SemaphoreType.DMA,
            pltpu.SemaphoreType.DMA,
        ),
    )
    def _sc_build(dst_hbm, src_hbm, c_hbm, deg_hbm, acc, dega,
                  dbuf0, sbuf0, dbuf1, sbuf1, sem0, sem1):
        wid = lax.axis_index("s") * 2 + lax.axis_index("c")
        slots = ((dbuf0, sbuf0, sem0), (dbuf1, sbuf1, sem1))
        ones = jnp.ones((16,), jnp.float32)
        zeros = jnp.zeros((16,), jnp.float32)

        def start(gch, slot):
            ch = gch % NCH
            db, sb, sem = slots[slot]
            return (
                pltpu.async_copy(dst_hbm.at[pl.ds(ch * CHUNK, CHUNK)],
                                 db, sem),
                pltpu.async_copy(src_hbm.at[pl.ds(ch * CHUNK, CHUNK)],
                                 sb, sem),
            )

        pending = {0: start(0, 0)}
        for p in range(BPW):
            blk = wid + NW * p
            base = blk * ROWS

            def zbody(j, _):
                def zrow(r, _):
                    acc[r, pl.ds(j * 16, 16)] = zeros
                    return 0

                return lax.fori_loop(0, ROWS, zrow, 0)

            lax.fori_loop(0, N // 16, zbody, 0)
            dega[pl.ds(0, 16)] = zeros
            dega[pl.ds(16, 16)] = zeros

            for ch in range(NCH):
                gch = p * NCH + ch
                slot = gch % 2
                for h in pending.pop(gch):
                    h.wait()
                if gch + 1 < BPW * NCH:
                    pending[gch + 1] = start(gch + 1, 1 - slot)
                db, sb, _ = slots[slot]

                @plsc.parallel_loop(0, CHUNK // 16, unroll=4)
                def ebody(i):
                    dv = db[pl.ds(i * 16, 16)]
                    sv = sb[pl.ds(i * 16, 16)]
                    loc = dv - base
                    m = (loc >= 0) & (loc < ROWS)
                    locc = lax.bitwise_and(loc, ROWS - 1)
                    plsc.addupdate_scatter(acc, [locc, sv], ones, mask=m)
                    plsc.addupdate_scatter(dega, [locc], ones, mask=m)

            for h in range(ROWS // 16):
                r = lax.iota(jnp.int32, 16) + h * 16
                plsc.addupdate_scatter(acc, [r, base + r], ones)

            pltpu.sync_copy(acc, c_hbm.at[pl.ds(base, ROWS)])
            pltpu.sync_copy(dega, deg_hbm.at[pl.ds(base, ROWS)])

    return _sc_build


BR = 256
_PREC = lax.Precision.HIGHEST


def _cdot(c_ref, t):
    return jnp.dot(c_ref[...], t, preferred_element_type=jnp.float32)


def _tc_body(x_ref, w0_ref, deg_ref, c_ref, w1_ref, b0_ref, b1_ref, out_ref):
    dis = lax.rsqrt(deg_ref[...] + 1.0)
    t0 = dis * jnp.dot(x_ref[...], w0_ref[...],
                       preferred_element_type=jnp.float32, precision=_PREC)
    h1 = jnp.maximum(dis * _cdot(c_ref, t0) + b0_ref[...], 0.0)
    t1 = jnp.dot(h1, w1_ref[...],
                 preferred_element_type=jnp.float32, precision=_PREC) * dis
    out_ref[...] = dis * _cdot(c_ref, t1) + b1_ref[...]


def _gcn_stack(x, C, deg, W0, b0, W1, b1):
    deg2 = deg.reshape(N, 1)
    return pl.pallas_call(
        _tc_body,
        out_shape=jax.ShapeDtypeStruct((N, D), jnp.float32),
        compiler_params=pltpu.CompilerParams(
            vmem_limit_bytes=100 * 1024 * 1024),
    )(x, W0, deg2, C, W1, b0.reshape(1, D), b1.reshape(1, D))


def kernel(x, edge_index, coordinates, W0, b0, W1, b1):
    ei = edge_index.astype(jnp.int32)
    src = ei[0]
    dst = ei[1]
    C, deg = _sc_build_fn()(dst, src)
    out = _gcn_stack(x, C, deg, W0, b0, W1, b1)
    labels = jax.random.randint(jax.random.key(42), (x.shape[0],), 0, 3)
    return (out, labels)


# speedup vs baseline: 18.5062x; 1.0152x over previous
import functools

import jax
import jax.numpy as jnp
from jax import lax
from jax.experimental import pallas as pl
from jax.experimental.pallas import tpu as pltpu
from jax.experimental.pallas import tpu_sc as plsc

N = 2048
D = 128
E = 65536
NW = 32
ROWS = 32
NBLK = N // ROWS
BPW = NBLK // NW
CHUNK = 8192
NCH = E // CHUNK


@functools.cache
def _sc_build_fn():
    mesh = plsc.VectorSubcoreMesh(core_axis_name="c", subcore_axis_name="s")

    @functools.partial(
        pl.kernel,
        out_type=jax.ShapeDtypeStruct((N, N), jnp.float32),
        mesh=mesh,
        compiler_params=pltpu.CompilerParams(needs_layout_passes=False),
        scratch_types=(
            pltpu.VMEM((ROWS, N), jnp.float32),
            pltpu.VMEM((CHUNK,), jnp.int32),
            pltpu.VMEM((CHUNK,), jnp.int32),
            pltpu.VMEM((CHUNK,), jnp.int32),
            pltpu.VMEM((CHUNK,), jnp.int32),
            pltpu.SemaphoreType.DMA,
            pltpu.SemaphoreType.DMA,
        ),
    )
    def _sc_build(dst_hbm, src_hbm, c_hbm, acc,
                  dbuf0, sbuf0, dbuf1, sbuf1, sem0, sem1):
        wid = lax.axis_index("s") * 2 + lax.axis_index("c")
        slots = ((dbuf0, sbuf0, sem0), (dbuf1, sbuf1, sem1))
        ones = jnp.ones((16,), jnp.float32)
        zeros = jnp.zeros((16,), jnp.float32)

        def start(gch, slot):
            ch = gch % NCH
            db, sb, sem = slots[slot]
            return (
                pltpu.async_copy(dst_hbm.at[pl.ds(ch * CHUNK, CHUNK)],
                                 db, sem),
                pltpu.async_copy(src_hbm.at[pl.ds(ch * CHUNK, CHUNK)],
                                 sb, sem),
            )

        pending = {0: start(0, 0)}
        for p in range(BPW):
            blk = wid + NW * p
            base = blk * ROWS

            def zbody(j, _):
                def zrow(r, _):
                    acc[r, pl.ds(j * 16, 16)] = zeros
                    return 0

                return lax.fori_loop(0, ROWS, zrow, 0)

            lax.fori_loop(0, N // 16, zbody, 0)

            for ch in range(NCH):
                gch = p * NCH + ch
                slot = gch % 2
                for h in pending.pop(gch):
                    h.wait()
                if gch + 1 < BPW * NCH:
                    pending[gch + 1] = start(gch + 1, 1 - slot)
                db, sb, _ = slots[slot]

                @plsc.parallel_loop(0, CHUNK // 16, unroll=8)
                def ebody(i):
                    dv = db[pl.ds(i * 16, 16)]
                    sv = sb[pl.ds(i * 16, 16)]
                    loc = dv - base
                    m = (loc >= 0) & (loc < ROWS)
                    locc = lax.bitwise_and(loc, ROWS - 1)
                    plsc.addupdate_scatter(acc, [locc, sv], ones, mask=m)

            for h in range(ROWS // 16):
                r = lax.iota(jnp.int32, 16) + h * 16
                plsc.addupdate_scatter(acc, [r, base + r], ones)

            pltpu.sync_copy(acc, c_hbm.at[pl.ds(base, ROWS)])

    return _sc_build


BR = 256
_PREC = lax.Precision.HIGHEST


def _cdot(c_ref, t):
    return jnp.dot(c_ref[...], t, preferred_element_type=jnp.float32)


def _tc_body(x_ref, w0_ref, c_ref, w1_ref, b0_ref, b1_ref, out_ref):
    deg = jnp.sum(c_ref[...], axis=1, keepdims=True)
    dis = lax.rsqrt(deg)
    t0 = dis * jnp.dot(x_ref[...], w0_ref[...],
                       preferred_element_type=jnp.float32, precision=_PREC)
    h1 = jnp.maximum(dis * _cdot(c_ref, t0) + b0_ref[...], 0.0)
    t1 = jnp.dot(h1, w1_ref[...],
                 preferred_element_type=jnp.float32, precision=_PREC) * dis
    out_ref[...] = dis * _cdot(c_ref, t1) + b1_ref[...]


def _gcn_stack(x, C, W0, b0, W1, b1):
    return pl.pallas_call(
        _tc_body,
        out_shape=jax.ShapeDtypeStruct((N, D), jnp.float32),
        compiler_params=pltpu.CompilerParams(
            vmem_limit_bytes=100 * 1024 * 1024),
    )(x, W0, C, W1, b0.reshape(1, D), b1.reshape(1, D))


def kernel(x, edge_index, coordinates, W0, b0, W1, b1):
    ei = edge_index.astype(jnp.int32)
    src = ei[0]
    dst = ei[1]
    C = _sc_build_fn()(dst, src)
    out = _gcn_stack(x, C, W0, b0, W1, b1)
    labels = jax.random.randint(jax.random.key(42), (x.shape[0],), 0, 3)
    return (out, labels)


# speedup vs baseline: 24.6009x; 1.3293x over previous
import functools

import jax
import jax.numpy as jnp
from jax import lax
from jax.experimental import pallas as pl
from jax.experimental.pallas import tpu as pltpu
from jax.experimental.pallas import tpu_sc as plsc

N = 2048
D = 128
E = 65536
NW = 32
ROWS = 32
NBLK = N // ROWS
BPW = NBLK // NW
CHUNK = 8192
NCH = E // CHUNK


@functools.cache
def _sc_build_fn():
    mesh = plsc.VectorSubcoreMesh(core_axis_name="c", subcore_axis_name="s")

    @functools.partial(
        pl.kernel,
        out_type=jax.ShapeDtypeStruct((N, N), jnp.float32),
        mesh=mesh,
        compiler_params=pltpu.CompilerParams(needs_layout_passes=False),
        scratch_types=(
            pltpu.VMEM((ROWS, N), jnp.float32),
            pltpu.VMEM((CHUNK,), jnp.int32),
            pltpu.VMEM((CHUNK,), jnp.int32),
            pltpu.VMEM((CHUNK,), jnp.int32),
            pltpu.VMEM((CHUNK,), jnp.int32),
            pltpu.SemaphoreType.DMA,
            pltpu.SemaphoreType.DMA,
        ),
    )
    def _sc_build(dst_hbm, src_hbm, c_hbm, acc,
                  dbuf0, sbuf0, dbuf1, sbuf1, sem0, sem1):
        wid = lax.axis_index("s") * 2 + lax.axis_index("c")
        slots = ((dbuf0, sbuf0, sem0), (dbuf1, sbuf1, sem1))
        ones = jnp.ones((16,), jnp.float32)
        zeros = jnp.zeros((16,), jnp.float32)

        def start(gch, slot):
            ch = gch % NCH
            db, sb, sem = slots[slot]
            return (
                pltpu.async_copy(dst_hbm.at[pl.ds(ch * CHUNK, CHUNK)],
                                 db, sem),
                pltpu.async_copy(src_hbm.at[pl.ds(ch * CHUNK, CHUNK)],
                                 sb, sem),
            )

        pending = {0: start(0, 0)}
        for p in range(BPW):
            blk = wid + NW * p
            base = blk * ROWS

            @plsc.parallel_loop(0, N // 16, unroll=2)
            def zbody(j):
                for r in range(ROWS):
                    acc[r, pl.ds(j * 16, 16)] = zeros

            for ch in range(NCH):
                gch = p * NCH + ch
                slot = gch % 2
                for h in pending.pop(gch):
                    h.wait()
                if gch + 1 < BPW * NCH:
                    pending[gch + 1] = start(gch + 1, 1 - slot)
                db, sb, _ = slots[slot]

                @plsc.parallel_loop(0, CHUNK // 16, unroll=8)
                def ebody(i):
                    dv = db[pl.ds(i * 16, 16)]
                    sv = sb[pl.ds(i * 16, 16)]
                    loc = dv - base
                    m = (loc >= 0) & (loc < ROWS)
                    locc = lax.bitwise_and(loc, ROWS - 1)
                    plsc.addupdate_scatter(acc, [locc, sv], ones, mask=m)

            for h in range(ROWS // 16):
                r = lax.iota(jnp.int32, 16) + h * 16
                plsc.addupdate_scatter(acc, [r, base + r], ones)

            pltpu.sync_copy(acc, c_hbm.at[pl.ds(base, ROWS)])

    return _sc_build


BR = 256
_PREC = lax.Precision.HIGHEST


def _cdot(c_ref, t):
    return jnp.dot(c_ref[...], t, preferred_element_type=jnp.float32)


def _tc_body(x_ref, w0_ref, c_ref, w1_ref, b0_ref, b1_ref, out_ref):
    deg = jnp.sum(c_ref[...], axis=1, keepdims=True)
    dis = lax.rsqrt(deg)
    t0 = dis * jnp.dot(x_ref[...], w0_ref[...],
                       preferred_element_type=jnp.float32, precision=_PREC)
    h1 = jnp.maximum(dis * _cdot(c_ref, t0) + b0_ref[...], 0.0)
    t1 = jnp.dot(h1, w1_ref[...],
                 preferred_element_type=jnp.float32, precision=_PREC) * dis
    out_ref[...] = dis * _cdot(c_ref, t1) + b1_ref[...]


def _gcn_stack(x, C, W0, b0, W1, b1):
    return pl.pallas_call(
        _tc_body,
        out_shape=jax.ShapeDtypeStruct((N, D), jnp.float32),
        compiler_params=pltpu.CompilerParams(
            vmem_limit_bytes=100 * 1024 * 1024),
    )(x, W0, C, W1, b0.reshape(1, D), b1.reshape(1, D))


def kernel(x, edge_index, coordinates, W0, b0, W1, b1):
    ei = edge_index.astype(jnp.int32)
    src = ei[0]
    dst = ei[1]
    C = _sc_build_fn()(dst, src)
    out = _gcn_stack(x, C, W0, b0, W1, b1)
    labels = jax.random.randint(jax.random.key(42), (x.shape[0],), 0, 3)
    return (out, labels)
